# baseline (speedup 1.0000x reference)
import jax
import jax.numpy as jnp
from jax.experimental import pallas as pl
from jax.experimental.pallas import tpu as pltpu

_T = 64
_BT = 8


def _loss_step(lab_ref, x_ref, out_ref):
    i = pl.program_id(0)
    x = x_ref[...]
    s = jnp.sum(x, axis=-1)
    d = s - 1.0
    part = jnp.sum(d * d)
    lab = lab_ref[0, 0, :]
    c_idx = jax.lax.broadcasted_iota(jnp.int32, s.shape, 1)
    mask = c_idx == lab[:, None]
    corr = jnp.sum(jnp.where(mask, 99.0 - 18.0 * s, 0.0))
    acc = (part + corr) * (0.5 / _T)

    @pl.when(i == 0)
    def _init():
        out_ref[...] = acc.reshape(1, 1)

    @pl.when(i != 0)
    def _accum():
        out_ref[...] += acc.reshape(1, 1)


def kernel(outputs, labels):
    B, C, H, W, T = outputs.shape
    x = outputs.reshape(B, C, T)
    n_steps = B // _BT
    lab3 = labels.reshape(n_steps, 1, _BT)
    out = pl.pallas_call(
        _loss_step,
        grid=(n_steps,),
        in_specs=[
            pl.BlockSpec((1, 1, _BT), lambda i: (i, 0, 0)),
            pl.BlockSpec((_BT, C, T), lambda i: (i, 0, 0)),
        ],
        out_specs=pl.BlockSpec((1, 1), lambda i: (0, 0)),
        out_shape=jax.ShapeDtypeStruct((1, 1), jnp.float32),
    )(lab3, x)
    return out[0, 0]
